# baseline (speedup 1.0000x reference)
import functools

import jax
import jax.numpy as jnp
from jax import lax
from jax.experimental import pallas as pl
from jax.experimental.pallas import tpu as pltpu
from jax.experimental.pallas import tpu_sc as plsc

N = 10000
E = 160000
L = 20000
D = 256
H = 128
NS = 16
CK = 128
EROWS = E // CK
CH_LO = 78
CH_HI = 80
NBUF = 3
CBUF = 4
LP = 20480
_MESH = plsc.VectorSubcoreMesh(core_axis_name="c", subcore_axis_name="s",
                               num_cores=2, num_subcores=16)

_f32 = jnp.float32


def _zero_fill(buf, nrow, ncol):
    z = jnp.zeros((16,), _f32)

    def body(i, _):
        r = i // (ncol // 16)
        col = (i % (ncol // 16)) * 16
        buf[r, pl.ds(col, 16)] = z
        return 0

    lax.fori_loop(0, nrow * (ncol // 16), body, 0)


def _zero_accum(accum, zero_v, s):
    @pl.when(s < 15)
    def _():
        for j in range(5):
            pltpu.sync_copy(zero_v, accum.at[pl.ds(s * 640 + j * 128, 128)])

    @pl.when(s == 15)
    def _():
        for j in range(3):
            pltpu.sync_copy(zero_v, accum.at[pl.ds(9600 + j * 128, 128)])
        pltpu.sync_copy(zero_v.at[pl.ds(0, 16)], accum.at[pl.ds(9984, 16)])


def _writeback(accum, out_hbm, s):
    @pl.when(s < 15)
    def _():
        pltpu.sync_copy(accum.at[pl.ds(s * 640, 640)], out_hbm.at[pl.ds(s * 640, 640)])

    @pl.when(s == 15)
    def _():
        pltpu.sync_copy(accum.at[pl.ds(9600, 400)], out_hbm.at[pl.ds(9600, 400)])


@functools.partial(
    pl.kernel,
    out_type=[jax.ShapeDtypeStruct((N, 128), _f32), jax.ShapeDtypeStruct((N, 128), _f32)],
    mesh=_MESH,
    scratch_types=[
        pltpu.VMEM((CK, 128), _f32),
        pltpu.VMEM((64, 1, 256), jnp.int32),
        pltpu.VMEM((CH_HI, 1, CK), jnp.int32),
        pltpu.SemaphoreType.DMA,
        pltpu.SemaphoreType.DMA,
        pltpu.SemaphoreType.DMA,
        pltpu.SemaphoreType.DMA,
        pltpu.VMEM_SHARED((N, 128), _f32),
    ],
)
def _counts(didx3, sidx3, pair_e, out_e, out_d,
            e0_v, dummy_v, idx_v, sm0, sm1, sm2, sm3, accum):
    c = lax.axis_index("c")
    s = lax.axis_index("s")
    sems = [sm0, sm1, sm2, sm3]
    _zero_fill(e0_v, CK, 128)
    _zero_accum(accum, e0_v, s)
    lane = lax.broadcasted_iota(jnp.int32, (16,), 0)
    e0 = jnp.where(lane == 0, 1.0, 0.0).astype(_f32)

    def fill_e0(i, _):
        e0_v[i, pl.ds(0, 16)] = e0
        return 0

    lax.fori_loop(0, CK, fill_e0, 0)
    plsc.subcore_barrier()

    def drain(b):
        pltpu.make_async_copy(pair_e.at[pl.ds(0, 64)], dummy_v, sems[b]).wait()

    def run(eidx3, out_hbm, nchunks, row0):
        pltpu.sync_copy(eidx3.at[pl.ds(row0, nchunks)], idx_v.at[pl.ds(0, nchunks)])
        nss = nchunks // CBUF

        def superstep(ss, _):
            for b in range(CBUF):
                g = ss * CBUF + b

                @pl.when(ss > 0)
                def _():
                    drain(b)

                pltpu.async_copy(e0_v, accum.at[idx_v.at[g, 0]], sems[b], add=True)
            return 0

        lax.fori_loop(0, nss, superstep, 0)
        for b in range(nchunks - nss * CBUF):
            g = nss * CBUF + b
            drain(b)
            pltpu.async_copy(e0_v, accum.at[idx_v.at[g, 0]], sems[b], add=True)
        for b in range(CBUF):
            drain(b)

    def percore(eidx3, out_hbm):
        @pl.when(s < 15)
        def _():
            run(eidx3, out_hbm, CH_LO, s * CH_LO)

        @pl.when(s == 15)
        def _():
            run(eidx3, out_hbm, CH_HI, 15 * CH_LO)

        plsc.subcore_barrier()
        _writeback(accum, out_hbm, s)

    pl.when(c == 0)(lambda: percore(didx3, out_e))
    pl.when(c == 1)(lambda: percore(sidx3, out_d))


@functools.partial(
    pl.kernel,
    out_type=[jax.ShapeDtypeStruct((N, H), _f32), jax.ShapeDtypeStruct((N, H), _f32)],
    mesh=_MESH,
    scratch_types=[
        pltpu.VMEM((CK, H), _f32),
        pltpu.VMEM((CK, H), _f32),
        pltpu.VMEM((CK, H), _f32),
        pltpu.VMEM((1, 1, 256), jnp.int32),
        pltpu.VMEM((1, 1, 256), jnp.int32),
        pltpu.VMEM((1, 1, 256), jnp.int32),
        pltpu.VMEM((1, 1, 256), jnp.int32),
        pltpu.VMEM((1, 1, 256), jnp.int32),
        pltpu.VMEM((1, 1, 256), jnp.int32),
        pltpu.SemaphoreType.DMA,
        pltpu.SemaphoreType.DMA,
        pltpu.SemaphoreType.DMA,
        pltpu.SemaphoreType.DMA,
        pltpu.SemaphoreType.DMA,
        pltpu.SemaphoreType.DMA,
        pltpu.SemaphoreType.DMA,
        pltpu.SemaphoreType.DMA,
        pltpu.SemaphoreType.DMA,
        pltpu.SemaphoreType.DMA,
        pltpu.SemaphoreType.DMA,
        pltpu.SemaphoreType.DMA,
        pltpu.VMEM_SHARED((N, H), _f32),
    ],
)
def _segsum(featA, featB, pair3, outA, outB,
            r0, r1, r2, i00, i01, i02, i10, i11, i12,
            g0, g1, g2, t0, t1, t2, s00, s01, s02, s10, s11, s12, accum):
    c = lax.axis_index("c")
    s = lax.axis_index("s")
    rows = [r0, r1, r2]
    ib = [[i00, i01, i02], [i10, i11, i12]]
    semg = [g0, g1, g2]
    semt = [t0, t1, t2]
    semi = [[s00, s01, s02], [s10, s11, s12]]
    _zero_fill(r0, CK, H)
    _zero_accum(accum, r0, s)
    plsc.subcore_barrier()

    def edge_pipeline(feat_hbm, nchunks, row0):
        def wait_g(b):
            pltpu.make_async_copy(feat_hbm.at[pl.ds(0, CK)], rows[b], semg[b]).wait()

        def wait_t(b):
            pltpu.make_async_copy(feat_hbm.at[pl.ds(0, CK)], rows[b], semt[b]).wait()

        def load_idx(q, b, g):
            pltpu.async_copy(pair3.at[pl.ds(row0 + g, 1)], ib[q][b], semi[q][b])

        def wait_i(q, b):
            pltpu.make_async_copy(pair3.at[pl.ds(0, 1)], ib[q][b], semi[q][b]).wait()

        def gather(q, b, g):
            pltpu.async_copy(feat_hbm.at[ib[q][b].at[0, 0, pl.ds(0, CK)]],
                             rows[b], semg[b])

        def scatter(q, b):
            pltpu.async_copy(rows[b], accum.at[ib[q][b].at[0, 0, pl.ds(CK, CK)]],
                             semt[b], add=True)

        for g in range(6):
            load_idx(g // 3, g % 3, g)
        for b in range(NBUF):
            wait_i(0, b)
            gather(0, b, b)

        def visit(g, q, b):
            wait_g(b)
            scatter(q, b)
            gp3 = g + 3

            @pl.when(gp3 < nchunks)
            def _():
                wait_t(b)
                gp6 = g + 6

                @pl.when(gp6 < nchunks)
                def _():
                    load_idx(q, b, gp6)

                wait_i(1 - q, b)
                gather(1 - q, b, gp3)

        def superstep(ss, _):
            for k in range(6):
                visit(ss * 6 + k, k // 3, k % 3)
            return 0

        lax.fori_loop(0, nchunks // 6, superstep, 0)
        base = (nchunks // 6) * 6
        for k in range(nchunks - base):
            g = base + k
            q = (g // 3) % 2
            b = g % 3
            wait_g(b)
            scatter(q, b)
        for b in range(NBUF):
            wait_t(b)

    def percore(feat_hbm, out_hbm):
        @pl.when(s < 15)
        def _():
            edge_pipeline(feat_hbm, CH_LO, s * CH_LO)

        @pl.when(s == 15)
        def _():
            edge_pipeline(feat_hbm, CH_HI, 15 * CH_LO)

        plsc.subcore_barrier()
        _writeback(accum, out_hbm, s)

    pl.when(c == 0)(lambda: percore(featA, outA))
    pl.when(c == 1)(lambda: percore(featB, outB))


GCK = 64
GCHT = LP // (32 * GCK)


@functools.partial(
    pl.kernel,
    out_type=[jax.ShapeDtypeStruct((LP, D), _f32), jax.ShapeDtypeStruct((LP, D), _f32)],
    mesh=_MESH,
    scratch_types=[
        pltpu.VMEM((GCHT, 1, GCK), jnp.int32),
        pltpu.VMEM((GCHT, 1, GCK), jnp.int32),
        pltpu.VMEM((GCK, D), _f32),
        pltpu.VMEM((GCK, D), _f32),
        pltpu.VMEM((GCK, D), _f32),
        pltpu.VMEM((GCK, D), _f32),
        pltpu.SemaphoreType.DMA,
        pltpu.SemaphoreType.DMA,
        pltpu.SemaphoreType.DMA,
        pltpu.SemaphoreType.DMA,
        pltpu.SemaphoreType.DMA,
        pltpu.SemaphoreType.DMA,
        pltpu.SemaphoreType.DMA,
        pltpu.SemaphoreType.DMA,
    ],
)
def _gather2(pd_hbm, pe_hbm, row3, col3, out1, out2,
             ridx_v, cidx_v, a0, a1, b0, b1,
             ga0, ga1, gb0, gb1, wa0, wa1, wb0, wb1):
    c = lax.axis_index("c")
    s = lax.axis_index("s")
    w = s * 2 + c
    bufA = [a0, a1]
    bufB = [b0, b1]
    ga = [ga0, ga1]
    gb = [gb0, gb1]
    wa = [wa0, wa1]
    wb = [wb0, wb1]
    base = w * (GCHT * GCK)
    pltpu.sync_copy(row3.at[pl.ds(w * GCHT, GCHT)], ridx_v)
    pltpu.sync_copy(col3.at[pl.ds(w * GCHT, GCHT)], cidx_v)

    def wait_gathers(q):
        pltpu.make_async_copy(pd_hbm.at[pl.ds(0, GCK)], bufA[q], ga[q]).wait()
        pltpu.make_async_copy(pd_hbm.at[pl.ds(0, GCK)], bufB[q], gb[q]).wait()

    def wait_writebacks(q):
        pltpu.make_async_copy(pd_hbm.at[pl.ds(0, GCK)], bufA[q], wa[q]).wait()
        pltpu.make_async_copy(pd_hbm.at[pl.ds(0, GCK)], bufB[q], wb[q]).wait()

    def gathers(g, q):
        pltpu.async_copy(pd_hbm.at[ridx_v.at[g, 0]], bufA[q], ga[q])
        pltpu.async_copy(pe_hbm.at[cidx_v.at[g, 0]], bufB[q], gb[q])

    gathers(0, 0)

    def visit(ss, g, q):
        gp = g + 1
        q2 = 1 - q

        @pl.when(gp < GCHT)
        def _():
            @pl.when(ss + q > 0)
            def _():
                wait_writebacks(q2)

            gathers(gp, q2)

        wait_gathers(q)
        off = base + g * GCK
        pltpu.async_copy(bufA[q], out1.at[pl.ds(off, GCK)], wa[q])
        pltpu.async_copy(bufB[q], out2.at[pl.ds(off, GCK)], wb[q])

    def superstep(ss, _):
        for q in range(2):
            visit(ss, ss * 2 + q, q)
        return 0

    lax.fori_loop(0, GCHT // 2, superstep, 0)
    for q in range(2):
        wait_writebacks(q)


_RB = 1000


def _lin_body(a0, a1, cnt, x0, x1, Wl0, Wl1, Wr0, Wr1, b, o0, o1, *, relu):
    inv = 1.0 / jnp.maximum(cnt[...], 1.0)
    acc = jnp.dot(a0[...] * inv, Wl0[...], preferred_element_type=_f32)
    acc = acc + jnp.dot(a1[...] * inv, Wl1[...], preferred_element_type=_f32)
    acc = acc + jnp.dot(x0[...], Wr0[...], preferred_element_type=_f32)
    acc = acc + jnp.dot(x1[...], Wr1[...], preferred_element_type=_f32)
    acc = acc + b[...]
    if relu:
        acc = jnp.maximum(acc, 0.0)
    o0[...] = acc[:, :H]
    o1[...] = acc[:, H:]


def _make_linear(relu, interpret=False):
    wspec = pl.BlockSpec((H, D), lambda i: (0, 0))
    return pl.pallas_call(
        functools.partial(_lin_body, relu=relu),
        grid=(N // _RB,),
        interpret=interpret,
        in_specs=[
            pl.BlockSpec((_RB, H), lambda i: (i, 0)),
            pl.BlockSpec((_RB, H), lambda i: (i, 0)),
            pl.BlockSpec((_RB, 1), lambda i: (i, 0)),
            pl.BlockSpec((_RB, H), lambda i: (i, 0)),
            pl.BlockSpec((_RB, H), lambda i: (i, 0)),
            wspec, wspec, wspec, wspec,
            pl.BlockSpec((1, D), lambda i: (0, 0)),
        ],
        out_specs=[
            pl.BlockSpec((_RB, H), lambda i: (i, 0)),
            pl.BlockSpec((_RB, H), lambda i: (i, 0)),
        ],
        out_shape=[
            jax.ShapeDtypeStruct((N, H), _f32),
            jax.ShapeDtypeStruct((N, H), _f32),
        ],
    )


_linear_relu = _make_linear(True)
_linear_id = _make_linear(False)


def _mm2_body(xd0, xd1, xe0, xe1, Wa0, Wa1, Wb0, Wb1, od, oe):
    od[...] = (jnp.dot(xd0[...], Wa0[...], preferred_element_type=_f32)
               + jnp.dot(xd1[...], Wa1[...], preferred_element_type=_f32))
    oe[...] = (jnp.dot(xe0[...], Wb0[...], preferred_element_type=_f32)
               + jnp.dot(xe1[...], Wb1[...], preferred_element_type=_f32))


def _make_mm2(interpret=False):
    xspec = pl.BlockSpec((_RB, H), lambda i: (i, 0))
    wspec = pl.BlockSpec((H, D), lambda i: (0, 0))
    return pl.pallas_call(
        _mm2_body,
        grid=(N // _RB,),
        interpret=interpret,
        in_specs=[xspec, xspec, xspec, xspec, wspec, wspec, wspec, wspec],
        out_specs=[
            pl.BlockSpec((_RB, D), lambda i: (i, 0)),
            pl.BlockSpec((_RB, D), lambda i: (i, 0)),
        ],
        out_shape=[
            jax.ShapeDtypeStruct((N, D), _f32),
            jax.ShapeDtypeStruct((N, D), _f32),
        ],
    )


_mm2 = _make_mm2()


def _dec_body(g1, g2, b1, w2r, b2, o):
    z = jnp.maximum(g1[...] + g2[...] + b1[...], 0.0)
    o[...] = jnp.sum(z * w2r[...], axis=1, keepdims=True) + b2[...]


_DRB = 1024


def _make_dec(interpret=False):
    return pl.pallas_call(
        _dec_body,
        grid=(LP // _DRB,),
        interpret=interpret,
        in_specs=[
            pl.BlockSpec((_DRB, D), lambda i: (i, 0)),
            pl.BlockSpec((_DRB, D), lambda i: (i, 0)),
            pl.BlockSpec((1, D), lambda i: (0, 0)),
            pl.BlockSpec((1, D), lambda i: (0, 0)),
            pl.BlockSpec((1, 1), lambda i: (0, 0)),
        ],
        out_specs=pl.BlockSpec((_DRB, 1), lambda i: (i, 0)),
        out_shape=jax.ShapeDtypeStruct((LP, 1), _f32),
    )


_dec = _make_dec()


def kernel(x_drug, x_effect, edge_index, edge_label_index,
           Wl1_dwe, bl1_dwe, Wr1_dwe, Wl1_ewd, bl1_ewd, Wr1_ewd,
           Wl2_dwe, bl2_dwe, Wr2_dwe, Wl2_ewd, bl2_ewd, Wr2_ewd,
           W_dec1, b_dec1, W_dec2, b_dec2):
    xd0, xd1 = x_drug[:, :H], x_drug[:, H:]
    xe0, xe1 = x_effect[:, :H], x_effect[:, H:]
    src2 = edge_index[0].reshape(EROWS, CK)
    dst2 = edge_index[1].reshape(EROWS, CK)
    pair_e = jnp.concatenate([src2, dst2], axis=1).reshape(EROWS, 1, 2 * CK)
    pair_d = jnp.concatenate([dst2, src2], axis=1).reshape(EROWS, 1, 2 * CK)
    cnt_e16, cnt_d16 = _counts(dst2.reshape(EROWS, 1, CK), src2.reshape(EROWS, 1, CK), pair_e)
    cnt_e = cnt_e16[:, 0:1]
    cnt_d = cnt_d16[:, 0:1]

    aE0, aE1 = _segsum(xd0, xd1, pair_e)
    aD0, aD1 = _segsum(xe0, xe1, pair_d)
    he0, he1 = _linear_relu(aE0, aE1, cnt_e, xe0, xe1,
                            Wl1_dwe[:H], Wl1_dwe[H:], Wr1_dwe[:H], Wr1_dwe[H:],
                            bl1_dwe[None])
    hd0, hd1 = _linear_relu(aD0, aD1, cnt_d, xd0, xd1,
                            Wl1_ewd[:H], Wl1_ewd[H:], Wr1_ewd[:H], Wr1_ewd[H:],
                            bl1_ewd[None])

    bE0, bE1 = _segsum(hd0, hd1, pair_e)
    bD0, bD1 = _segsum(he0, he1, pair_d)
    ze0, ze1 = _linear_id(bE0, bE1, cnt_e, he0, he1,
                          Wl2_dwe[:H], Wl2_dwe[H:], Wr2_dwe[:H], Wr2_dwe[H:],
                          bl2_dwe[None])
    zd0, zd1 = _linear_id(bD0, bD1, cnt_d, hd0, hd1,
                          Wl2_ewd[:H], Wl2_ewd[H:], Wr2_ewd[:H], Wr2_ewd[H:],
                          bl2_ewd[None])

    p_d, p_e = _mm2(zd0, zd1, ze0, ze1, W_dec1[:H], W_dec1[H:D],
                    W_dec1[D:D + H], W_dec1[D + H:])
    pad = jnp.zeros((LP - L,), edge_label_index.dtype)
    row3 = jnp.concatenate([edge_label_index[0], pad]).reshape(LP // GCK, 1, GCK)
    col3 = jnp.concatenate([edge_label_index[1], pad]).reshape(LP // GCK, 1, GCK)
    g1, g2 = _gather2(p_d, p_e, row3, col3)
    pre = _dec(g1, g2, b_dec1[None], W_dec2.T, b_dec2[None])[:L, 0]

    z_d = jnp.concatenate([zd0, zd1], axis=1)
    z_e = jnp.concatenate([ze0, ze1], axis=1)
    return pre, z_d, z_e


# speedup vs baseline: 1.0070x; 1.0070x over previous
import functools

import jax
import jax.numpy as jnp
from jax import lax
from jax.experimental import pallas as pl
from jax.experimental.pallas import tpu as pltpu
from jax.experimental.pallas import tpu_sc as plsc

N = 10000
E = 160000
L = 20000
D = 256
H = 128
NS = 16
CK = 128
EROWS = E // CK
CH_LO = 78
CH_HI = 80
NBUF = 3
CBUF = 4
LP = 20480
_MESH = plsc.VectorSubcoreMesh(core_axis_name="c", subcore_axis_name="s",
                               num_cores=2, num_subcores=16)

_f32 = jnp.float32


def _zero_fill(buf, nrow, ncol):
    z = jnp.zeros((16,), _f32)

    def body(i, _):
        r = i // (ncol // 16)
        col = (i % (ncol // 16)) * 16
        buf[r, pl.ds(col, 16)] = z
        return 0

    lax.fori_loop(0, nrow * (ncol // 16), body, 0)


def _zero_accum(accum, zero_v, s):
    @pl.when(s < 15)
    def _():
        for j in range(5):
            pltpu.sync_copy(zero_v, accum.at[pl.ds(s * 640 + j * 128, 128)])

    @pl.when(s == 15)
    def _():
        for j in range(3):
            pltpu.sync_copy(zero_v, accum.at[pl.ds(9600 + j * 128, 128)])
        pltpu.sync_copy(zero_v.at[pl.ds(0, 16)], accum.at[pl.ds(9984, 16)])


def _writeback(accum, out_hbm, s):
    @pl.when(s < 15)
    def _():
        pltpu.sync_copy(accum.at[pl.ds(s * 640, 640)], out_hbm.at[pl.ds(s * 640, 640)])

    @pl.when(s == 15)
    def _():
        pltpu.sync_copy(accum.at[pl.ds(9600, 400)], out_hbm.at[pl.ds(9600, 400)])


@functools.partial(
    pl.kernel,
    out_type=[jax.ShapeDtypeStruct((N, 128), _f32), jax.ShapeDtypeStruct((N, 128), _f32)],
    mesh=_MESH,
    scratch_types=[
        pltpu.VMEM((CK, 128), _f32),
        pltpu.VMEM((64, 1, 256), jnp.int32),
        pltpu.VMEM((CH_HI, 1, CK), jnp.int32),
        pltpu.SemaphoreType.DMA,
        pltpu.SemaphoreType.DMA,
        pltpu.SemaphoreType.DMA,
        pltpu.SemaphoreType.DMA,
        pltpu.VMEM_SHARED((N, 128), _f32),
    ],
)
def _counts(didx3, sidx3, pair_e, out_e, out_d,
            e0_v, dummy_v, idx_v, sm0, sm1, sm2, sm3, accum):
    c = lax.axis_index("c")
    s = lax.axis_index("s")
    sems = [sm0, sm1, sm2, sm3]
    _zero_fill(e0_v, CK, 128)
    _zero_accum(accum, e0_v, s)
    lane = lax.broadcasted_iota(jnp.int32, (16,), 0)
    e0 = jnp.where(lane == 0, 1.0, 0.0).astype(_f32)

    def fill_e0(i, _):
        e0_v[i, pl.ds(0, 16)] = e0
        return 0

    lax.fori_loop(0, CK, fill_e0, 0)
    plsc.subcore_barrier()

    def drain(b):
        pltpu.make_async_copy(pair_e.at[pl.ds(0, 64)], dummy_v, sems[b]).wait()

    def run(eidx3, out_hbm, nchunks, row0):
        pltpu.sync_copy(eidx3.at[pl.ds(row0, nchunks)], idx_v.at[pl.ds(0, nchunks)])
        nss = nchunks // CBUF

        def superstep(ss, _):
            for b in range(CBUF):
                g = ss * CBUF + b

                @pl.when(ss > 0)
                def _():
                    drain(b)

                pltpu.async_copy(e0_v, accum.at[idx_v.at[g, 0]], sems[b], add=True)
            return 0

        lax.fori_loop(0, nss, superstep, 0)
        for b in range(nchunks - nss * CBUF):
            g = nss * CBUF + b
            drain(b)
            pltpu.async_copy(e0_v, accum.at[idx_v.at[g, 0]], sems[b], add=True)
        for b in range(CBUF):
            drain(b)

    def percore(eidx3, out_hbm):
        @pl.when(s < 15)
        def _():
            run(eidx3, out_hbm, CH_LO, s * CH_LO)

        @pl.when(s == 15)
        def _():
            run(eidx3, out_hbm, CH_HI, 15 * CH_LO)

        plsc.subcore_barrier()
        _writeback(accum, out_hbm, s)

    pl.when(c == 0)(lambda: percore(didx3, out_e))
    pl.when(c == 1)(lambda: percore(sidx3, out_d))


@functools.partial(
    pl.kernel,
    out_type=[jax.ShapeDtypeStruct((N, H), _f32), jax.ShapeDtypeStruct((N, H), _f32)],
    mesh=_MESH,
    scratch_types=[
        pltpu.VMEM((CK, H), _f32),
        pltpu.VMEM((CK, H), _f32),
        pltpu.VMEM((CK, H), _f32),
        pltpu.VMEM((1, 1, 256), jnp.int32),
        pltpu.VMEM((1, 1, 256), jnp.int32),
        pltpu.VMEM((1, 1, 256), jnp.int32),
        pltpu.VMEM((1, 1, 256), jnp.int32),
        pltpu.VMEM((1, 1, 256), jnp.int32),
        pltpu.VMEM((1, 1, 256), jnp.int32),
        pltpu.SemaphoreType.DMA,
        pltpu.SemaphoreType.DMA,
        pltpu.SemaphoreType.DMA,
        pltpu.SemaphoreType.DMA,
        pltpu.SemaphoreType.DMA,
        pltpu.SemaphoreType.DMA,
        pltpu.SemaphoreType.DMA,
        pltpu.SemaphoreType.DMA,
        pltpu.SemaphoreType.DMA,
        pltpu.SemaphoreType.DMA,
        pltpu.SemaphoreType.DMA,
        pltpu.SemaphoreType.DMA,
        pltpu.VMEM_SHARED((N, H), _f32),
    ],
)
def _segsum(featA, featB, pair3, outA, outB,
            r0, r1, r2, i00, i01, i02, i10, i11, i12,
            g0, g1, g2, t0, t1, t2, s00, s01, s02, s10, s11, s12, accum):
    c = lax.axis_index("c")
    s = lax.axis_index("s")
    rows = [r0, r1, r2]
    ib = [[i00, i01, i02], [i10, i11, i12]]
    semg = [g0, g1, g2]
    semt = [t0, t1, t2]
    semi = [[s00, s01, s02], [s10, s11, s12]]
    _zero_fill(r0, CK, H)
    _zero_accum(accum, r0, s)
    plsc.subcore_barrier()

    def edge_pipeline(feat_hbm, nchunks, row0):
        def wait_g(b):
            pltpu.make_async_copy(feat_hbm.at[pl.ds(0, CK)], rows[b], semg[b]).wait()

        def wait_t(b):
            pltpu.make_async_copy(feat_hbm.at[pl.ds(0, CK)], rows[b], semt[b]).wait()

        def load_idx(q, b, g):
            pltpu.async_copy(pair3.at[pl.ds(row0 + g, 1)], ib[q][b], semi[q][b])

        def wait_i(q, b):
            pltpu.make_async_copy(pair3.at[pl.ds(0, 1)], ib[q][b], semi[q][b]).wait()

        def gather(q, b, g):
            pltpu.async_copy(feat_hbm.at[ib[q][b].at[0, 0, pl.ds(0, CK)]],
                             rows[b], semg[b])

        def scatter(q, b):
            pltpu.async_copy(rows[b], accum.at[ib[q][b].at[0, 0, pl.ds(CK, CK)]],
                             semt[b], add=True)

        for g in range(6):
            load_idx(g // 3, g % 3, g)
        for b in range(NBUF):
            wait_i(0, b)
            gather(0, b, b)

        def visit(g, q, b):
            wait_g(b)
            scatter(q, b)
            gp3 = g + 3

            @pl.when(gp3 < nchunks)
            def _():
                wait_t(b)
                gp6 = g + 6

                @pl.when(gp6 < nchunks)
                def _():
                    load_idx(q, b, gp6)

                wait_i(1 - q, b)
                gather(1 - q, b, gp3)

        def superstep(ss, _):
            for k in range(6):
                visit(ss * 6 + k, k // 3, k % 3)
            return 0

        lax.fori_loop(0, nchunks // 6, superstep, 0)
        base = (nchunks // 6) * 6
        for k in range(nchunks - base):
            g = base + k
            q = (g // 3) % 2
            b = g % 3
            wait_g(b)
            scatter(q, b)
        for b in range(NBUF):
            wait_t(b)

    def percore(feat_hbm, out_hbm):
        @pl.when(s < 15)
        def _():
            edge_pipeline(feat_hbm, CH_LO, s * CH_LO)

        @pl.when(s == 15)
        def _():
            edge_pipeline(feat_hbm, CH_HI, 15 * CH_LO)

        plsc.subcore_barrier()
        _writeback(accum, out_hbm, s)

    pl.when(c == 0)(lambda: percore(featA, outA))
    pl.when(c == 1)(lambda: percore(featB, outB))


GCK = 64
GCHT = LP // (32 * GCK)


@functools.partial(
    pl.kernel,
    out_type=[jax.ShapeDtypeStruct((LP, D), _f32), jax.ShapeDtypeStruct((LP, D), _f32)],
    mesh=_MESH,
    scratch_types=[
        pltpu.VMEM((GCHT, 1, GCK), jnp.int32),
        pltpu.VMEM((GCHT, 1, GCK), jnp.int32),
        pltpu.VMEM((GCK, D), _f32),
        pltpu.VMEM((GCK, D), _f32),
        pltpu.VMEM((GCK, D), _f32),
        pltpu.VMEM((GCK, D), _f32),
        pltpu.SemaphoreType.DMA,
        pltpu.SemaphoreType.DMA,
        pltpu.SemaphoreType.DMA,
        pltpu.SemaphoreType.DMA,
        pltpu.SemaphoreType.DMA,
        pltpu.SemaphoreType.DMA,
        pltpu.SemaphoreType.DMA,
        pltpu.SemaphoreType.DMA,
    ],
)
def _gather2(pd_hbm, pe_hbm, row3, col3, out1, out2,
             ridx_v, cidx_v, a0, a1, b0, b1,
             ga0, ga1, gb0, gb1, wa0, wa1, wb0, wb1):
    c = lax.axis_index("c")
    s = lax.axis_index("s")
    w = s * 2 + c
    bufA = [a0, a1]
    bufB = [b0, b1]
    ga = [ga0, ga1]
    gb = [gb0, gb1]
    wa = [wa0, wa1]
    wb = [wb0, wb1]
    base = w * (GCHT * GCK)
    pltpu.sync_copy(row3.at[pl.ds(w * GCHT, GCHT)], ridx_v)
    pltpu.sync_copy(col3.at[pl.ds(w * GCHT, GCHT)], cidx_v)

    def wait_gathers(q):
        pltpu.make_async_copy(pd_hbm.at[pl.ds(0, GCK)], bufA[q], ga[q]).wait()
        pltpu.make_async_copy(pd_hbm.at[pl.ds(0, GCK)], bufB[q], gb[q]).wait()

    def wait_writebacks(q):
        pltpu.make_async_copy(pd_hbm.at[pl.ds(0, GCK)], bufA[q], wa[q]).wait()
        pltpu.make_async_copy(pd_hbm.at[pl.ds(0, GCK)], bufB[q], wb[q]).wait()

    def gathers(g, q):
        pltpu.async_copy(pd_hbm.at[ridx_v.at[g, 0]], bufA[q], ga[q])
        pltpu.async_copy(pe_hbm.at[cidx_v.at[g, 0]], bufB[q], gb[q])

    gathers(0, 0)

    def visit(ss, g, q):
        gp = g + 1
        q2 = 1 - q

        @pl.when(gp < GCHT)
        def _():
            @pl.when(ss + q > 0)
            def _():
                wait_writebacks(q2)

            gathers(gp, q2)

        wait_gathers(q)
        off = base + g * GCK
        pltpu.async_copy(bufA[q], out1.at[pl.ds(off, GCK)], wa[q])
        pltpu.async_copy(bufB[q], out2.at[pl.ds(off, GCK)], wb[q])

    def superstep(ss, _):
        for q in range(2):
            visit(ss, ss * 2 + q, q)
        return 0

    lax.fori_loop(0, GCHT // 2, superstep, 0)
    for q in range(2):
        wait_writebacks(q)


_RB = 1000


def _lin_body(a0, a1, cnt, x0, x1, Wl0, Wl1, Wr0, Wr1, b, o0, o1, *, relu):
    inv = 1.0 / jnp.maximum(cnt[...], 1.0)
    acc = jnp.dot(a0[...] * inv, Wl0[...], preferred_element_type=_f32)
    acc = acc + jnp.dot(a1[...] * inv, Wl1[...], preferred_element_type=_f32)
    acc = acc + jnp.dot(x0[...], Wr0[...], preferred_element_type=_f32)
    acc = acc + jnp.dot(x1[...], Wr1[...], preferred_element_type=_f32)
    acc = acc + b[...]
    if relu:
        acc = jnp.maximum(acc, 0.0)
    o0[...] = acc[:, :H]
    o1[...] = acc[:, H:]


def _make_linear(relu, interpret=False):
    wspec = pl.BlockSpec((H, D), lambda i: (0, 0))
    return pl.pallas_call(
        functools.partial(_lin_body, relu=relu),
        grid=(N // _RB,),
        interpret=interpret,
        in_specs=[
            pl.BlockSpec((_RB, H), lambda i: (i, 0)),
            pl.BlockSpec((_RB, H), lambda i: (i, 0)),
            pl.BlockSpec((_RB, 1), lambda i: (i, 0)),
            pl.BlockSpec((_RB, H), lambda i: (i, 0)),
            pl.BlockSpec((_RB, H), lambda i: (i, 0)),
            wspec, wspec, wspec, wspec,
            pl.BlockSpec((1, D), lambda i: (0, 0)),
        ],
        out_specs=[
            pl.BlockSpec((_RB, H), lambda i: (i, 0)),
            pl.BlockSpec((_RB, H), lambda i: (i, 0)),
        ],
        out_shape=[
            jax.ShapeDtypeStruct((N, H), _f32),
            jax.ShapeDtypeStruct((N, H), _f32),
        ],
    )


_linear_relu = _make_linear(True)
_linear_id = _make_linear(False)


def _mm2_body(x0, x1, W0, W1, o):
    acc = jnp.dot(x0[...], W0[...], preferred_element_type=_f32)
    o[...] = acc + jnp.dot(x1[...], W1[...], preferred_element_type=_f32)


def _make_mm2(interpret=False):
    return pl.pallas_call(
        _mm2_body,
        grid=(N // _RB,),
        interpret=interpret,
        in_specs=[
            pl.BlockSpec((_RB, H), lambda i: (i, 0)),
            pl.BlockSpec((_RB, H), lambda i: (i, 0)),
            pl.BlockSpec((H, D), lambda i: (0, 0)),
            pl.BlockSpec((H, D), lambda i: (0, 0)),
        ],
        out_specs=pl.BlockSpec((_RB, D), lambda i: (i, 0)),
        out_shape=jax.ShapeDtypeStruct((N, D), _f32),
    )


_mm2 = _make_mm2()


def _dec_body(g1, g2, b1, w2r, b2, o):
    z = jnp.maximum(g1[...] + g2[...] + b1[...], 0.0)
    o[...] = jnp.sum(z * w2r[...], axis=1, keepdims=True) + b2[...]


_DRB = 1024


def _make_dec(interpret=False):
    return pl.pallas_call(
        _dec_body,
        grid=(LP // _DRB,),
        interpret=interpret,
        in_specs=[
            pl.BlockSpec((_DRB, D), lambda i: (i, 0)),
            pl.BlockSpec((_DRB, D), lambda i: (i, 0)),
            pl.BlockSpec((1, D), lambda i: (0, 0)),
            pl.BlockSpec((1, D), lambda i: (0, 0)),
            pl.BlockSpec((1, 1), lambda i: (0, 0)),
        ],
        out_specs=pl.BlockSpec((_DRB, 1), lambda i: (i, 0)),
        out_shape=jax.ShapeDtypeStruct((LP, 1), _f32),
    )


_dec = _make_dec()


def kernel(x_drug, x_effect, edge_index, edge_label_index,
           Wl1_dwe, bl1_dwe, Wr1_dwe, Wl1_ewd, bl1_ewd, Wr1_ewd,
           Wl2_dwe, bl2_dwe, Wr2_dwe, Wl2_ewd, bl2_ewd, Wr2_ewd,
           W_dec1, b_dec1, W_dec2, b_dec2):
    xd0, xd1 = x_drug[:, :H], x_drug[:, H:]
    xe0, xe1 = x_effect[:, :H], x_effect[:, H:]
    src2 = edge_index[0].reshape(EROWS, CK)
    dst2 = edge_index[1].reshape(EROWS, CK)
    pair_e = jnp.concatenate([src2, dst2], axis=1).reshape(EROWS, 1, 2 * CK)
    pair_d = jnp.concatenate([dst2, src2], axis=1).reshape(EROWS, 1, 2 * CK)
    cnt_e16, cnt_d16 = _counts(dst2.reshape(EROWS, 1, CK), src2.reshape(EROWS, 1, CK), pair_e)
    cnt_e = cnt_e16[:, 0:1]
    cnt_d = cnt_d16[:, 0:1]

    aE0, aE1 = _segsum(xd0, xd1, pair_e)
    aD0, aD1 = _segsum(xe0, xe1, pair_d)
    he0, he1 = _linear_relu(aE0, aE1, cnt_e, xe0, xe1,
                            Wl1_dwe[:H], Wl1_dwe[H:], Wr1_dwe[:H], Wr1_dwe[H:],
                            bl1_dwe[None])
    hd0, hd1 = _linear_relu(aD0, aD1, cnt_d, xd0, xd1,
                            Wl1_ewd[:H], Wl1_ewd[H:], Wr1_ewd[:H], Wr1_ewd[H:],
                            bl1_ewd[None])

    bE0, bE1 = _segsum(hd0, hd1, pair_e)
    bD0, bD1 = _segsum(he0, he1, pair_d)
    ze0, ze1 = _linear_id(bE0, bE1, cnt_e, he0, he1,
                          Wl2_dwe[:H], Wl2_dwe[H:], Wr2_dwe[:H], Wr2_dwe[H:],
                          bl2_dwe[None])
    zd0, zd1 = _linear_id(bD0, bD1, cnt_d, hd0, hd1,
                          Wl2_ewd[:H], Wl2_ewd[H:], Wr2_ewd[:H], Wr2_ewd[H:],
                          bl2_ewd[None])

    p_d = _mm2(zd0, zd1, W_dec1[:H], W_dec1[H:D])
    p_e = _mm2(ze0, ze1, W_dec1[D:D + H], W_dec1[D + H:])
    pad = jnp.zeros((LP - L,), edge_label_index.dtype)
    row3 = jnp.concatenate([edge_label_index[0], pad]).reshape(LP // GCK, 1, GCK)
    col3 = jnp.concatenate([edge_label_index[1], pad]).reshape(LP // GCK, 1, GCK)
    g1, g2 = _gather2(p_d, p_e, row3, col3)
    pre = _dec(g1, g2, b_dec1[None], W_dec2.T, b_dec2[None])[:L, 0]

    z_d = jnp.concatenate([zd0, zd1], axis=1)
    z_e = jnp.concatenate([ze0, ze1], axis=1)
    return pre, z_d, z_e
